# speedup vs baseline: 1.2240x; 1.2240x over previous
import functools

import jax
import jax.numpy as jnp
from jax import lax
from jax.experimental import pallas as pl
from jax.experimental.pallas import tpu as pltpu
from jax.experimental.pallas import tpu_sc as plsc

VOCAB = 100000
EMBED_DIM = 128
BATCH = 4096
FIELDS = 26

TOT = BATCH * FIELDS
NUM_CORES = 2
NUM_SUBCORES = 16
NW = NUM_CORES * NUM_SUBCORES
B_PER_W = TOT // NW
CHUNK = 256
N_CHUNKS = B_PER_W // CHUNK

_mesh = plsc.VectorSubcoreMesh(core_axis_name="c", subcore_axis_name="s")


@functools.partial(
    pl.kernel,
    mesh=_mesh,
    out_type=jax.ShapeDtypeStruct((TOT, EMBED_DIM), jnp.float32),
    scratch_types=[
        pltpu.VMEM((B_PER_W,), jnp.int32),
        pltpu.VMEM((CHUNK, EMBED_DIM), jnp.float32),
        pltpu.SemaphoreType.DMA,
    ],
)
def _embedding_gather(idx_hbm, table_hbm, out_hbm, idx_v, rows_v, sem):
    wid = lax.axis_index("s") * NUM_CORES + lax.axis_index("c")
    base = wid * B_PER_W
    pltpu.sync_copy(idx_hbm.at[pl.ds(base, B_PER_W)], idx_v)
    for c in range(N_CHUNKS):
        off = c * CHUNK
        pltpu.async_copy(
            table_hbm.at[idx_v.at[pl.ds(off, CHUNK)]], rows_v, sem
        ).wait()
        pltpu.sync_copy(rows_v, out_hbm.at[pl.ds(base + off, CHUNK)])


def kernel(input, weight):
    idx = input.reshape(-1).astype(jnp.int32)
    out = _embedding_gather(idx, weight)
    return out.reshape(BATCH, FIELDS, EMBED_DIM)


# speedup vs baseline: 1.2932x; 1.0566x over previous
import functools

import jax
import jax.numpy as jnp
from jax import lax
from jax.experimental import pallas as pl
from jax.experimental.pallas import tpu as pltpu
from jax.experimental.pallas import tpu_sc as plsc

VOCAB = 100000
EMBED_DIM = 128
BATCH = 4096
FIELDS = 26

TOT = BATCH * FIELDS
NUM_CORES = 2
NUM_SUBCORES = 16
NW = NUM_CORES * NUM_SUBCORES
B_PER_W = TOT // NW
CHUNK = 256
N_CHUNKS = B_PER_W // CHUNK

_mesh = plsc.VectorSubcoreMesh(core_axis_name="c", subcore_axis_name="s")


@functools.partial(
    pl.kernel,
    mesh=_mesh,
    out_type=jax.ShapeDtypeStruct((TOT, EMBED_DIM), jnp.float32),
    scratch_types=[
        pltpu.VMEM((B_PER_W,), jnp.int32),
        pltpu.VMEM((CHUNK, EMBED_DIM), jnp.float32),
        pltpu.VMEM((CHUNK, EMBED_DIM), jnp.float32),
        pltpu.SemaphoreType.DMA,
        pltpu.SemaphoreType.DMA,
        pltpu.SemaphoreType.DMA,
        pltpu.SemaphoreType.DMA,
    ],
)
def _embedding_gather(
    idx_hbm, table_hbm, out_hbm, idx_v, rows0, rows1, gs0, gs1, os0, os1
):
    wid = lax.axis_index("s") * NUM_CORES + lax.axis_index("c")
    base = wid * B_PER_W
    rows = (rows0, rows1)
    gsem = (gs0, gs1)
    osem = (os0, os1)

    pltpu.sync_copy(idx_hbm.at[pl.ds(base, B_PER_W)], idx_v)

    def gather_start(c):
        b = c % 2
        return pltpu.async_copy(
            table_hbm.at[idx_v.at[pl.ds(c * CHUNK, CHUNK)]], rows[b], gsem[b]
        )

    def writeback_start(c):
        b = c % 2
        return pltpu.async_copy(
            rows[b], out_hbm.at[pl.ds(base + c * CHUNK, CHUNK)], osem[b]
        )

    g = gather_start(0)
    wb = [None, None]
    for c in range(N_CHUNKS):
        g_cur = g
        if c + 1 < N_CHUNKS:
            nb = (c + 1) % 2
            if wb[nb] is not None:
                wb[nb].wait()
            g = gather_start(c + 1)
        g_cur.wait()
        wb[c % 2] = writeback_start(c)
    for h in wb:
        if h is not None:
            h.wait()


def kernel(input, weight):
    idx = input.reshape(-1).astype(jnp.int32)
    out = _embedding_gather(idx, weight)
    return out.reshape(BATCH, FIELDS, EMBED_DIM)


# speedup vs baseline: 1.2975x; 1.0033x over previous
import functools

import jax
import jax.numpy as jnp
from jax import lax
from jax.experimental import pallas as pl
from jax.experimental.pallas import tpu as pltpu
from jax.experimental.pallas import tpu_sc as plsc

VOCAB = 100000
EMBED_DIM = 128
BATCH = 4096
FIELDS = 26

TOT = BATCH * FIELDS
NUM_CORES = 2
NUM_SUBCORES = 16
NW = NUM_CORES * NUM_SUBCORES
B_PER_W = TOT // NW
CHUNK = 208
N_CHUNKS = B_PER_W // CHUNK
NBUF = 4

_mesh = plsc.VectorSubcoreMesh(core_axis_name="c", subcore_axis_name="s")

_scratch = (
    [pltpu.VMEM((B_PER_W,), jnp.int32)]
    + [pltpu.VMEM((CHUNK, EMBED_DIM), jnp.float32) for _ in range(NBUF)]
    + [pltpu.SemaphoreType.DMA for _ in range(2 * NBUF)]
)


@functools.partial(
    pl.kernel,
    mesh=_mesh,
    out_type=jax.ShapeDtypeStruct((TOT, EMBED_DIM), jnp.float32),
    scratch_types=_scratch,
)
def _embedding_gather(idx_hbm, table_hbm, out_hbm, idx_v, *bufs):
    rows = bufs[:NBUF]
    gsem = bufs[NBUF : 2 * NBUF]
    osem = bufs[2 * NBUF : 3 * NBUF]
    wid = lax.axis_index("s") * NUM_CORES + lax.axis_index("c")
    base = wid * B_PER_W

    pltpu.sync_copy(idx_hbm.at[pl.ds(base, B_PER_W)], idx_v)

    def gather_start(c):
        b = c % NBUF
        return pltpu.async_copy(
            table_hbm.at[idx_v.at[pl.ds(c * CHUNK, CHUNK)]], rows[b], gsem[b]
        )

    def writeback_start(c):
        b = c % NBUF
        return pltpu.async_copy(
            rows[b], out_hbm.at[pl.ds(base + c * CHUNK, CHUNK)], osem[b]
        )

    g = [None] * N_CHUNKS
    wb = [None] * NBUF
    for c in range(min(NBUF - 1, N_CHUNKS)):
        g[c] = gather_start(c)
    for c in range(N_CHUNKS):
        nxt = c + NBUF - 1
        if nxt < N_CHUNKS:
            b = nxt % NBUF
            if wb[b] is not None:
                wb[b].wait()
            g[nxt] = gather_start(nxt)
        g[c].wait()
        wb[c % NBUF] = writeback_start(c)
    for h in wb:
        if h is not None:
            h.wait()


def kernel(input, weight):
    idx = input.reshape(-1).astype(jnp.int32)
    out = _embedding_gather(idx, weight)
    return out.reshape(BATCH, FIELDS, EMBED_DIM)


# speedup vs baseline: 2.0101x; 1.5492x over previous
import functools

import jax
import jax.numpy as jnp
from jax import lax
from jax.experimental import pallas as pl
from jax.experimental.pallas import tpu as pltpu
from jax.experimental.pallas import tpu_sc as plsc

VOCAB = 100000
EMBED_DIM = 128
BATCH = 4096
FIELDS = 26
FIELDS_PAD = 32

NUM_CORES = 2
NUM_SUBCORES = 16
NW = NUM_CORES * NUM_SUBCORES
BATCH_PER_W = BATCH // NW
CHUNKB = 8
N_CHUNKS = BATCH_PER_W // CHUNKB
SLAB_ROWS = CHUNKB * FIELDS_PAD
NBUF = 3

_mesh = plsc.VectorSubcoreMesh(core_axis_name="c", subcore_axis_name="s")

_scratch = (
    [pltpu.VMEM((BATCH_PER_W * FIELDS_PAD,), jnp.int32)]
    + [pltpu.VMEM((SLAB_ROWS, EMBED_DIM), jnp.float32) for _ in range(NBUF)]
    + [pltpu.SemaphoreType.DMA for _ in range(2 * NBUF)]
)


@functools.partial(
    pl.kernel,
    mesh=_mesh,
    out_type=jax.ShapeDtypeStruct((BATCH, FIELDS, EMBED_DIM), jnp.float32),
    scratch_types=_scratch,
)
def _embedding_gather(idx_hbm, table_hbm, out_hbm, idx_v, *bufs):
    slabs = bufs[:NBUF]
    gsem = bufs[NBUF : 2 * NBUF]
    osem = bufs[2 * NBUF : 3 * NBUF]
    wid = lax.axis_index("s") * NUM_CORES + lax.axis_index("c")
    batch0 = wid * BATCH_PER_W

    pltpu.sync_copy(
        idx_hbm.at[pl.ds(batch0 * FIELDS_PAD, BATCH_PER_W * FIELDS_PAD)], idx_v
    )

    def gathers_start(c):
        b = c % NBUF
        handles = []
        for i in range(CHUNKB):
            off = c * SLAB_ROWS + i * FIELDS_PAD
            handles.append(
                pltpu.async_copy(
                    table_hbm.at[idx_v.at[pl.ds(off, FIELDS)]],
                    slabs[b].at[pl.ds(i * FIELDS_PAD, FIELDS)],
                    gsem[b],
                )
            )
        return handles

    def writebacks_start(c):
        b = c % NBUF
        handles = []
        for i in range(CHUNKB):
            handles.append(
                pltpu.async_copy(
                    slabs[b].at[pl.ds(i * FIELDS_PAD, FIELDS)],
                    out_hbm.at[batch0 + c * CHUNKB + i],
                    osem[b],
                )
            )
        return handles

    g = [None] * N_CHUNKS
    wb = [None] * NBUF
    for c in range(min(NBUF - 1, N_CHUNKS)):
        g[c] = gathers_start(c)
    for c in range(N_CHUNKS):
        nxt = c + NBUF - 1
        if nxt < N_CHUNKS:
            b = nxt % NBUF
            if wb[b] is not None:
                for h in wb[b]:
                    h.wait()
            g[nxt] = gathers_start(nxt)
        for h in g[c]:
            h.wait()
        wb[c % NBUF] = writebacks_start(c)
    for hs in wb:
        if hs is not None:
            for h in hs:
                h.wait()


def kernel(input, weight):
    idx = input.astype(jnp.int32)
    idx = jnp.pad(idx, ((0, 0), (0, FIELDS_PAD - FIELDS))).reshape(-1)
    return _embedding_gather(idx, weight)


# speedup vs baseline: 3.7292x; 1.8552x over previous
import functools

import jax
import jax.numpy as jnp
from jax import lax
from jax.experimental import pallas as pl
from jax.experimental.pallas import tpu as pltpu
from jax.experimental.pallas import tpu_sc as plsc

VOCAB = 100000
EMBED_DIM = 128
BATCH = 4096
FIELDS = 26

TOT = BATCH * FIELDS
NUM_CORES = 2
NUM_SUBCORES = 16
NW = NUM_CORES * NUM_SUBCORES
B_PER_W = TOT // NW
CHUNK = 208
N_CHUNKS = B_PER_W // CHUNK
NBUF = 3

_mesh = plsc.VectorSubcoreMesh(core_axis_name="c", subcore_axis_name="s")

_scratch = (
    [pltpu.VMEM((B_PER_W,), jnp.int32)]
    + [pltpu.VMEM((CHUNK, EMBED_DIM), jnp.float32) for _ in range(NBUF)]
    + [pltpu.SemaphoreType.DMA for _ in range(2 * NBUF)]
)


@functools.partial(
    pl.kernel,
    mesh=_mesh,
    out_type=jax.ShapeDtypeStruct((TOT, EMBED_DIM), jnp.float32),
    scratch_types=_scratch,
)
def _embedding_gather(idx_hbm, table_hbm, out_hbm, idx_v, *bufs):
    rows = bufs[:NBUF]
    gsem = bufs[NBUF : 2 * NBUF]
    osem = bufs[2 * NBUF : 3 * NBUF]
    wid = lax.axis_index("s") * NUM_CORES + lax.axis_index("c")
    base = wid * B_PER_W

    pltpu.sync_copy(idx_hbm.at[pl.ds(base, B_PER_W)], idx_v)

    def gather_start(c):
        b = c % NBUF
        return pltpu.async_copy(
            table_hbm.at[idx_v.at[pl.ds(c * CHUNK, CHUNK)]], rows[b], gsem[b]
        )

    def writeback_start(c):
        b = c % NBUF
        return pltpu.async_copy(
            rows[b], out_hbm.at[pl.ds(base + c * CHUNK, CHUNK)], osem[b]
        )

    g = [None] * N_CHUNKS
    wb = [None] * NBUF
    for c in range(min(NBUF - 1, N_CHUNKS)):
        g[c] = gather_start(c)
    for c in range(N_CHUNKS):
        nxt = c + NBUF - 1
        if nxt < N_CHUNKS:
            b = nxt % NBUF
            if wb[b] is not None:
                wb[b].wait()
            g[nxt] = gather_start(nxt)
        g[c].wait()
        wb[c % NBUF] = writeback_start(c)
    for h in wb:
        if h is not None:
            h.wait()


def kernel(input, weight):
    idx = input.T.reshape(-1).astype(jnp.int32)
    out = _embedding_gather(idx, weight)
    return out.reshape(FIELDS, BATCH, EMBED_DIM).transpose(1, 0, 2)
